# Initial kernel scaffold; baseline (speedup 1.0000x reference)
import jax
import jax.numpy as jnp
from jax.experimental import pallas as pl


def kernel(z_e, dictionary):
    raise NotImplementedError("write your pallas kernel here")


# speedup vs baseline: 6.9680x; 6.9680x over previous
import functools

import jax
import jax.numpy as jnp
from jax.experimental import pallas as pl

NE = 512
ED = 256
K = 5
B = 8192
BLK = 256
HI = jax.lax.Precision.HIGHEST


def _prep_body(d_ref, dn_ref, g_ref):
    d = d_ref[...]
    nrm = jnp.sqrt(jnp.sum(d * d, axis=0, keepdims=True))
    dn = d / nrm
    dn_ref[...] = dn
    dnb = dn.astype(jnp.bfloat16)
    g_ref[...] = jax.lax.dot_general(
        dnb, dnb, (((0,), (0,)), ((), ())), preferred_element_type=jnp.float32)


def _omp_body(zt_ref, dn_ref, g_ref, x_ref, zst_ref, loss_ref):
    z = zt_ref[...]
    dn = dn_ref[...]
    g = g_ref[...]

    hb = jax.lax.dot_general(z.astype(jnp.bfloat16), dn.astype(jnp.bfloat16),
                             (((1,), (0,)), ((), ())),
                             preferred_element_type=jnp.float32)
    iota = jax.lax.broadcasted_iota(jnp.int32, (BLK, NE), 1)
    mask = jnp.ones((BLK, NE), jnp.float32)
    h = hb
    idxs = []
    grows = []
    hbsel = []
    L = {(0, 0): 1.0}
    xs = []

    for k in range(K):
        a = jnp.abs(h) * mask
        m = jnp.max(a, axis=1, keepdims=True)
        idx = jnp.min(jnp.where(a == m, iota, NE), axis=1, keepdims=True)
        e = (iota == idx).astype(jnp.float32)
        mask = mask * (1.0 - e)
        hbsel.append(jnp.sum(hb * e, axis=1, keepdims=True))
        grow = jax.lax.dot_general(e, g, (((1,), (0,)), ((), ())),
                                   preferred_element_type=jnp.float32,
                                   precision=HI)
        if k > 0:
            gs = [jnp.sum(grows[j] * e, axis=1, keepdims=True)
                  for j in range(k)]
            w = []
            for i in range(k):
                acc = gs[i]
                for j in range(i):
                    acc = acc - L[(i, j)] * w[j]
                w.append(acc / L[(i, i)])
            ssq = w[0] * w[0]
            for j in range(1, k):
                ssq = ssq + w[j] * w[j]
            wc = jnp.sqrt(jnp.maximum(1.0 - ssq, 0.0))
            for j in range(k):
                L[(k, j)] = w[j]
            L[(k, k)] = wc
        idxs.append(idx)
        grows.append(grow)

        y = []
        for i in range(k + 1):
            acc = hbsel[i]
            for j in range(i):
                acc = acc - L[(i, j)] * y[j]
            y.append(acc / L[(i, i)])
        xs = [None] * (k + 1)
        for i in reversed(range(k + 1)):
            acc = y[i]
            for j in range(i + 1, k + 1):
                acc = acc - L[(j, i)] * xs[j]
            xs[i] = acc / L[(i, i)]

        def _bf(v):
            return v.astype(jnp.bfloat16).astype(jnp.float32)
        beta = _bf(xs[0]) * _bf(grows[0])
        for j in range(1, k + 1):
            beta = beta + _bf(xs[j]) * _bf(grows[j])
        h = hb - beta

    xdense = xs[0] * (iota == idxs[0]).astype(jnp.float32)
    for j in range(1, K):
        xdense = xdense + xs[j] * (iota == idxs[j]).astype(jnp.float32)
    x_ref[...] = xdense

    zdl = jax.lax.dot_general(xdense.astype(jnp.bfloat16),
                              dn.astype(jnp.bfloat16),
                              (((1,), (1,)), ((), ())),
                              preferred_element_type=jnp.float32)
    zst_ref[...] = z + (zdl - z)
    diff = zdl - z
    loss_ref[...] = jnp.full((1, 1, 128), jnp.sum(diff * diff), jnp.float32)


@jax.jit
def kernel(z_e, dictionary):
    nblk = B // BLK
    dn, g = pl.pallas_call(
        _prep_body,
        out_shape=(jax.ShapeDtypeStruct((ED, NE), jnp.float32),
                   jax.ShapeDtypeStruct((NE, NE), jnp.float32)),
    )(dictionary)

    zt = jnp.transpose(z_e, (0, 2, 3, 1)).reshape(ED, B).T

    x, zst, losses = pl.pallas_call(
        _omp_body,
        grid=(nblk,),
        in_specs=[
            pl.BlockSpec((BLK, ED), lambda b: (b, 0)),
            pl.BlockSpec((ED, NE), lambda b: (0, 0)),
            pl.BlockSpec((NE, NE), lambda b: (0, 0)),
        ],
        out_specs=(
            pl.BlockSpec((BLK, NE), lambda b: (b, 0)),
            pl.BlockSpec((BLK, ED), lambda b: (b, 0)),
            pl.BlockSpec((1, 1, 128), lambda b: (b, 0, 0)),
        ),
        out_shape=(
            jax.ShapeDtypeStruct((B, NE), jnp.float32),
            jax.ShapeDtypeStruct((B, ED), jnp.float32),
            jax.ShapeDtypeStruct((nblk, 1, 128), jnp.float32),
        ),
    )(zt, dn, g)

    coefficients = x.T
    z_st = jnp.transpose(zst.T.reshape(8, 32, 32, ED), (0, 3, 1, 2))
    loss = jnp.sum(losses[:, 0, 0]) * (1.0 + 0.25) / (8 * 32 * 32 * ED)
    return z_st, loss, coefficients
